# baseline (speedup 1.0000x reference)
import jax
import jax.numpy as jnp
from jax.experimental import pallas as pl
from jax.experimental.pallas import tpu as pltpu

_KB = 256


def _body(state_ref, w1_ref, b1_ref, w2_ref, b2_ref, w3_ref, b3_ref,
          v1_ref, vb1_ref, v2_ref, vb2_ref, v3_ref, vb3_ref, g_ref,
          probs_ref, value_ref, act_ref, alp_ref,
          sb, ha, hb, acc, lg):
    B, S = state_ref.shape
    A = b3_ref.shape[1]
    H = b1_ref.shape[1]
    nk = S // _KB
    o2 = nk
    o3 = 2 * nk
    o4 = 3 * nk
    o5 = 4 * nk
    o6 = 5 * nk

    i = pl.program_id(0)

    @pl.when(i == 0)
    def _cast_state():
        sb[...] = state_ref[...].astype(jnp.bfloat16)

    @pl.when(i < o2)
    def _l0():
        k = i
        part = jnp.dot(sb[:, pl.ds(k * _KB, _KB)],
                       w1_ref[...].astype(jnp.bfloat16),
                       preferred_element_type=jnp.float32)

        @pl.when(k == 0)
        def _():
            acc[...] = part

        @pl.when(k > 0)
        def _():
            acc[...] = acc[...] + part

        @pl.when(k == nk - 1)
        def _():
            ha[...] = jnp.tanh(acc[...] + b1_ref[...]).astype(jnp.bfloat16)

    @pl.when((i >= o2) & (i < o3))
    def _l1():
        k = i - o2
        part = jnp.dot(ha[:, pl.ds(k * _KB, _KB)],
                       w2_ref[...].astype(jnp.bfloat16),
                       preferred_element_type=jnp.float32)

        @pl.when(k == 0)
        def _():
            acc[...] = part

        @pl.when(k > 0)
        def _():
            acc[...] = acc[...] + part

        @pl.when(k == nk - 1)
        def _():
            hb[...] = jnp.tanh(acc[...] + b2_ref[...]).astype(jnp.bfloat16)

    @pl.when((i >= o3) & (i < o4))
    def _l2():
        k = i - o3
        part = jnp.dot(hb[:, pl.ds(k * _KB, _KB)],
                       w3_ref[...].astype(jnp.bfloat16),
                       preferred_element_type=jnp.float32)

        @pl.when(k == 0)
        def _():
            lg[...] = part

        @pl.when(k > 0)
        def _():
            lg[...] = lg[...] + part

    @pl.when((i >= o4) & (i < o5))
    def _l3():
        k = i - o4
        part = jnp.dot(sb[:, pl.ds(k * _KB, _KB)],
                       v1_ref[...].astype(jnp.bfloat16),
                       preferred_element_type=jnp.float32)

        @pl.when(k == 0)
        def _():
            acc[...] = part

        @pl.when(k > 0)
        def _():
            acc[...] = acc[...] + part

        @pl.when(k == nk - 1)
        def _():
            ha[...] = jnp.tanh(acc[...] + vb1_ref[...]).astype(jnp.bfloat16)

    @pl.when((i >= o5) & (i < o6))
    def _l4():
        k = i - o5
        part = jnp.dot(ha[:, pl.ds(k * _KB, _KB)],
                       v2_ref[...].astype(jnp.bfloat16),
                       preferred_element_type=jnp.float32)

        @pl.when(k == 0)
        def _():
            acc[...] = part

        @pl.when(k > 0)
        def _():
            acc[...] = acc[...] + part

        @pl.when(k == nk - 1)
        def _():
            hb[...] = jnp.tanh(acc[...] + vb2_ref[...]).astype(jnp.bfloat16)

    @pl.when(i == o6)
    def _fin():
        v3row = v3_ref[...].astype(jnp.bfloat16).astype(jnp.float32)
        hv = hb[...].astype(jnp.float32)
        value_ref[...] = (jnp.sum(hv * v3row, axis=-1, keepdims=True)
                          + vb3_ref[...])
        logits = lg[...] + b3_ref[...]
        m = jnp.max(logits, axis=-1, keepdims=True)
        e = jnp.exp(logits - m)
        p = e / jnp.sum(e, axis=-1, keepdims=True)
        probs_ref[...] = p
        lp = jnp.log(p + 1e-20)
        y = lp + g_ref[...]
        ym = jnp.max(y, axis=-1, keepdims=True)
        cols = jax.lax.broadcasted_iota(jnp.int32, (B, A), 1)
        idx = jnp.min(jnp.where(y == ym, cols, A), axis=-1, keepdims=True)
        act_ref[...] = idx
        alp_ref[...] = jnp.sum(jnp.where(cols == idx, lp, 0.0),
                               axis=-1, keepdims=True)


def kernel(state, W1, b1, W2, b2, W3, b3, V1, vb1, V2, vb2, V3, vb3):
    B, S = state.shape
    H = W1.shape[1]
    A = W3.shape[1]
    nk = S // _KB
    o2, o3, o4, o5, o6 = nk, 2 * nk, 3 * nk, 4 * nk, 5 * nk
    steps = o6 + 1

    g = jax.random.gumbel(jax.random.key(42), (B, A), jnp.float32)

    grid = (steps,)
    in_specs = [
        pl.BlockSpec((B, S), lambda i: (0, 0)),
        pl.BlockSpec((_KB, H), lambda i: (jnp.clip(i, 0, nk - 1), 0)),
        pl.BlockSpec((1, H), lambda i: (0, 0)),
        pl.BlockSpec((_KB, H), lambda i: (jnp.clip(i - o2, 0, nk - 1), 0)),
        pl.BlockSpec((1, H), lambda i: (0, 0)),
        pl.BlockSpec((_KB, A), lambda i: (jnp.clip(i - o3, 0, nk - 1), 0)),
        pl.BlockSpec((1, A), lambda i: (0, 0)),
        pl.BlockSpec((_KB, H), lambda i: (jnp.clip(i - o4, 0, nk - 1), 0)),
        pl.BlockSpec((1, H), lambda i: (0, 0)),
        pl.BlockSpec((_KB, H), lambda i: (jnp.clip(i - o5, 0, nk - 1), 0)),
        pl.BlockSpec((1, H), lambda i: (0, 0)),
        pl.BlockSpec((1, S), lambda i: (0, 0)),
        pl.BlockSpec((1, 1), lambda i: (0, 0)),
        pl.BlockSpec((B, A), lambda i: (0, 0)),
    ]
    out_specs = [
        pl.BlockSpec((B, A), lambda i: (0, 0)),
        pl.BlockSpec((B, 1), lambda i: (0, 0)),
        pl.BlockSpec((B, 1), lambda i: (0, 0)),
        pl.BlockSpec((B, 1), lambda i: (0, 0)),
    ]
    out_shape = [
        jax.ShapeDtypeStruct((B, A), jnp.float32),
        jax.ShapeDtypeStruct((B, 1), jnp.float32),
        jax.ShapeDtypeStruct((B, 1), jnp.int32),
        jax.ShapeDtypeStruct((B, 1), jnp.float32),
    ]
    scratch_shapes = [
        pltpu.VMEM((B, S), jnp.bfloat16),
        pltpu.VMEM((B, H), jnp.bfloat16),
        pltpu.VMEM((B, H), jnp.bfloat16),
        pltpu.VMEM((B, H), jnp.float32),
        pltpu.VMEM((B, A), jnp.float32),
    ]

    probs, value, act, alp = pl.pallas_call(
        _body,
        grid=grid,
        in_specs=in_specs,
        out_specs=out_specs,
        out_shape=out_shape,
        scratch_shapes=scratch_shapes,
    )(state, W1, b1.reshape(1, H), W2, b2.reshape(1, H),
      W3, b3.reshape(1, A), V1, vb1.reshape(1, H), V2, vb2.reshape(1, H),
      V3.reshape(1, S), vb3.reshape(1, 1), g)
    return probs, value, act[:, 0], alp[:, 0]


# speedup vs baseline: 1.0844x; 1.0844x over previous
import jax
import jax.numpy as jnp
from jax.experimental import pallas as pl
from jax.experimental.pallas import tpu as pltpu

_BN = 256
_AB = 256


def _body(state_ref, w1_ref, b1_ref, w2_ref, b2_ref, w3_ref, b3_ref,
          v1_ref, vb1_ref, v2_ref, vb2_ref, v3_ref, vb3_ref, g_ref,
          probs_ref, value_ref, act_ref, alp_ref,
          sb, ha, hb, lg):
    B, S = state_ref.shape
    A = b3_ref.shape[1]
    H = w1_ref.shape[0]
    nb = H // _BN
    na = pl.cdiv(A, _AB)
    o2 = nb
    o3 = 2 * nb
    o4 = o3 + na
    o5 = o4 + nb
    o6 = o5 + nb

    i = pl.program_id(0)

    @pl.when(i == 0)
    def _cast_state():
        sb[...] = state_ref[...].astype(jnp.bfloat16)

    @pl.when(i < o2)
    def _l0():
        j = i
        x = jnp.dot(sb[...], w1_ref[...],
                    preferred_element_type=jnp.float32)
        ha[:, pl.ds(j * _BN, _BN)] = jnp.tanh(x + b1_ref[...]).astype(jnp.bfloat16)

    @pl.when((i >= o2) & (i < o3))
    def _l1():
        j = i - o2
        x = jnp.dot(ha[...], w2_ref[...], preferred_element_type=jnp.float32)
        hb[:, pl.ds(j * _BN, _BN)] = jnp.tanh(x + b2_ref[...]).astype(jnp.bfloat16)

    @pl.when((i >= o3) & (i < o4))
    def _l2():
        j = i - o3
        lg[:, pl.ds(j * _AB, _AB)] = jnp.dot(
            hb[...], w3_ref[...], preferred_element_type=jnp.float32)

    @pl.when((i >= o4) & (i < o5))
    def _l3():
        j = i - o4
        x = jnp.dot(sb[...], v1_ref[...],
                    preferred_element_type=jnp.float32)
        ha[:, pl.ds(j * _BN, _BN)] = jnp.tanh(x + vb1_ref[...]).astype(jnp.bfloat16)

    @pl.when((i >= o5) & (i < o6))
    def _l4():
        j = i - o5
        x = jnp.dot(ha[...], v2_ref[...], preferred_element_type=jnp.float32)
        hb[:, pl.ds(j * _BN, _BN)] = jnp.tanh(x + vb2_ref[...]).astype(jnp.bfloat16)

    @pl.when(i == o6)
    def _fin():
        v3row = v3_ref[...].astype(jnp.bfloat16).astype(jnp.float32)
        hv = hb[...].astype(jnp.float32)
        value_ref[...] = (jnp.sum(hv * v3row, axis=-1, keepdims=True)
                          + vb3_ref[...])
        logits = lg[:, :A] + b3_ref[...]
        m = jnp.max(logits, axis=-1, keepdims=True)
        e = jnp.exp(logits - m)
        p = e / jnp.sum(e, axis=-1, keepdims=True)
        probs_ref[...] = p
        lp = jnp.log(p + 1e-20)
        y = lp + g_ref[...]
        ym = jnp.max(y, axis=-1, keepdims=True)
        cols = jax.lax.broadcasted_iota(jnp.int32, (B, A), 1)
        idx = jnp.min(jnp.where(y == ym, cols, A), axis=-1, keepdims=True)
        act_ref[...] = idx
        alp_ref[...] = jnp.sum(jnp.where(cols == idx, lp, 0.0),
                               axis=-1, keepdims=True)


def kernel(state, W1, b1, W2, b2, W3, b3, V1, vb1, V2, vb2, V3, vb3):
    B, S = state.shape
    H = W1.shape[1]
    A = W3.shape[1]
    nb = H // _BN
    na = pl.cdiv(A, _AB)
    Ap = na * _AB
    o2, o3 = nb, 2 * nb
    o4 = o3 + na
    o5 = o4 + nb
    o6 = o5 + nb
    steps = o6 + 1

    g = jax.random.gumbel(jax.random.key(42), (B, A), jnp.float32)

    grid = (steps,)
    in_specs = [
        pl.BlockSpec((B, S), lambda i: (0, 0)),
        pl.BlockSpec((S, _BN), lambda i: (0, jnp.clip(i, 0, nb - 1))),
        pl.BlockSpec((1, _BN), lambda i: (0, jnp.clip(i, 0, nb - 1))),
        pl.BlockSpec((H, _BN), lambda i: (0, jnp.clip(i - o2, 0, nb - 1))),
        pl.BlockSpec((1, _BN), lambda i: (0, jnp.clip(i - o2, 0, nb - 1))),
        pl.BlockSpec((S, _AB), lambda i: (0, jnp.clip(i - o3, 0, na - 1))),
        pl.BlockSpec((1, A), lambda i: (0, 0)),
        pl.BlockSpec((S, _BN), lambda i: (0, jnp.clip(i - o4, 0, nb - 1))),
        pl.BlockSpec((1, _BN), lambda i: (0, jnp.clip(i - o4, 0, nb - 1))),
        pl.BlockSpec((H, _BN), lambda i: (0, jnp.clip(i - o5, 0, nb - 1))),
        pl.BlockSpec((1, _BN), lambda i: (0, jnp.clip(i - o5, 0, nb - 1))),
        pl.BlockSpec((1, S), lambda i: (0, 0)),
        pl.BlockSpec((1, 1), lambda i: (0, 0)),
        pl.BlockSpec((B, A), lambda i: (0, 0)),
    ]
    out_specs = [
        pl.BlockSpec((B, A), lambda i: (0, 0)),
        pl.BlockSpec((B, 1), lambda i: (0, 0)),
        pl.BlockSpec((B, 1), lambda i: (0, 0)),
        pl.BlockSpec((B, 1), lambda i: (0, 0)),
    ]
    out_shape = [
        jax.ShapeDtypeStruct((B, A), jnp.float32),
        jax.ShapeDtypeStruct((B, 1), jnp.float32),
        jax.ShapeDtypeStruct((B, 1), jnp.int32),
        jax.ShapeDtypeStruct((B, 1), jnp.float32),
    ]
    scratch_shapes = [
        pltpu.VMEM((B, S), jnp.bfloat16),
        pltpu.VMEM((B, H), jnp.bfloat16),
        pltpu.VMEM((B, H), jnp.bfloat16),
        pltpu.VMEM((B, Ap), jnp.float32),
    ]

    probs, value, act, alp = pl.pallas_call(
        _body,
        grid=grid,
        in_specs=in_specs,
        out_specs=out_specs,
        out_shape=out_shape,
        scratch_shapes=scratch_shapes,
    )(state, W1, b1.reshape(1, H), W2, b2.reshape(1, H),
      W3, b3.reshape(1, A), V1, vb1.reshape(1, H), V2, vb2.reshape(1, H),
      V3.reshape(1, S), vb3.reshape(1, 1), g)
    return probs, value, act[:, 0], alp[:, 0]


# speedup vs baseline: 1.0990x; 1.0134x over previous
import jax
import jax.numpy as jnp
from jax.experimental import pallas as pl
from jax.experimental.pallas import tpu as pltpu

_BN = 256
_AB = 256
_NS = 4
_KS = 1024


def _body(state_ref,
          w1a, w1b, w1c, w1d, b1_ref,
          w2a, w2b, w2c, w2d, b2_ref,
          w3a, w3b, w3c, w3d, b3_ref,
          v1a, v1b, v1c, v1d, vb1_ref,
          v2a, v2b, v2c, v2d, vb2_ref,
          v3_ref, vb3_ref, g_ref,
          probs_ref, value_ref, act_ref, alp_ref,
          sb, ha, hb, lg):
    B, S = state_ref.shape
    A = b3_ref.shape[1]
    H = ha.shape[1]
    nb = H // _BN
    na = lg.shape[1] // _AB
    o2 = nb
    o3 = 2 * nb
    o4 = o3 + na
    o5 = o4 + nb
    o6 = o5 + nb

    i = pl.program_id(0)

    def _mm(lhs, parts):
        out = jnp.dot(lhs[:, pl.ds(0, _KS)], parts[0][...],
                      preferred_element_type=jnp.float32)
        for t in range(1, _NS):
            out = out + jnp.dot(lhs[:, pl.ds(t * _KS, _KS)], parts[t][...],
                                preferred_element_type=jnp.float32)
        return out

    @pl.when(i == 0)
    def _cast_state():
        sb[...] = state_ref[...].astype(jnp.bfloat16)

    @pl.when(i < o2)
    def _l0():
        j = i
        x = _mm(sb, (w1a, w1b, w1c, w1d))
        ha[:, pl.ds(j * _BN, _BN)] = jnp.tanh(x + b1_ref[...]).astype(jnp.bfloat16)

    @pl.when((i >= o2) & (i < o3))
    def _l1():
        j = i - o2
        x = _mm(ha, (w2a, w2b, w2c, w2d))
        hb[:, pl.ds(j * _BN, _BN)] = jnp.tanh(x + b2_ref[...]).astype(jnp.bfloat16)

    @pl.when((i >= o3) & (i < o4))
    def _l2():
        j = i - o3
        lg[:, pl.ds(j * _AB, _AB)] = _mm(hb, (w3a, w3b, w3c, w3d))

    @pl.when((i >= o4) & (i < o5))
    def _l3():
        j = i - o4
        x = _mm(sb, (v1a, v1b, v1c, v1d))
        ha[:, pl.ds(j * _BN, _BN)] = jnp.tanh(x + vb1_ref[...]).astype(jnp.bfloat16)

    @pl.when((i >= o5) & (i < o6))
    def _l4():
        j = i - o5
        x = _mm(ha, (v2a, v2b, v2c, v2d))
        hb[:, pl.ds(j * _BN, _BN)] = jnp.tanh(x + vb2_ref[...]).astype(jnp.bfloat16)

    @pl.when(i == o6)
    def _fin():
        v3row = v3_ref[...].astype(jnp.bfloat16).astype(jnp.float32)
        hv = hb[...].astype(jnp.float32)
        value_ref[...] = (jnp.sum(hv * v3row, axis=-1, keepdims=True)
                          + vb3_ref[...])
        logits = lg[:, :A] + b3_ref[...]
        m = jnp.max(logits, axis=-1, keepdims=True)
        e = jnp.exp(logits - m)
        p = e / jnp.sum(e, axis=-1, keepdims=True)
        probs_ref[...] = p
        lp = jnp.log(p + 1e-20)
        y = lp + g_ref[...]
        ym = jnp.max(y, axis=-1, keepdims=True)
        cols = jax.lax.broadcasted_iota(jnp.int32, (B, A), 1)
        idx = jnp.min(jnp.where(y == ym, cols, A), axis=-1, keepdims=True)
        act_ref[...] = idx
        alp_ref[...] = jnp.sum(jnp.where(cols == idx, lp, 0.0),
                               axis=-1, keepdims=True)


def kernel(state, W1, b1, W2, b2, W3, b3, V1, vb1, V2, vb2, V3, vb3):
    B, S = state.shape
    H = W1.shape[1]
    A = W3.shape[1]
    nb = H // _BN
    na = pl.cdiv(A, _AB)
    Ap = na * _AB
    o2, o3 = nb, 2 * nb
    o4 = o3 + na
    o5 = o4 + nb
    o6 = o5 + nb
    steps = o6 + 1

    with jax.ensure_compile_time_eval():
        g = jax.random.gumbel(jax.random.key(42), (B, A), jnp.float32)

    def sub_specs(bw, off, nblk):
        return [
            pl.BlockSpec((_KS, bw),
                         (lambda t: (lambda i: (t, jnp.clip(i - off, 0, nblk - 1))))(t))
            for t in range(_NS)
        ]

    in_specs = (
        [pl.BlockSpec((B, S), lambda i: (0, 0))]
        + sub_specs(_BN, 0, nb)
        + [pl.BlockSpec((1, _BN), lambda i: (0, jnp.clip(i, 0, nb - 1)))]
        + sub_specs(_BN, o2, nb)
        + [pl.BlockSpec((1, _BN), lambda i: (0, jnp.clip(i - o2, 0, nb - 1)))]
        + sub_specs(_AB, o3, na)
        + [pl.BlockSpec((1, A), lambda i: (0, 0))]
        + sub_specs(_BN, o4, nb)
        + [pl.BlockSpec((1, _BN), lambda i: (0, jnp.clip(i - o4, 0, nb - 1)))]
        + sub_specs(_BN, o5, nb)
        + [pl.BlockSpec((1, _BN), lambda i: (0, jnp.clip(i - o5, 0, nb - 1)))]
        + [
            pl.BlockSpec((1, S), lambda i: (0, 0)),
            pl.BlockSpec((1, 1), lambda i: (0, 0)),
            pl.BlockSpec((B, A), lambda i: (0, 0)),
        ]
    )
    out_specs = [
        pl.BlockSpec((B, A), lambda i: (0, 0)),
        pl.BlockSpec((B, 1), lambda i: (0, 0)),
        pl.BlockSpec((B, 1), lambda i: (0, 0)),
        pl.BlockSpec((B, 1), lambda i: (0, 0)),
    ]
    out_shape = [
        jax.ShapeDtypeStruct((B, A), jnp.float32),
        jax.ShapeDtypeStruct((B, 1), jnp.float32),
        jax.ShapeDtypeStruct((B, 1), jnp.int32),
        jax.ShapeDtypeStruct((B, 1), jnp.float32),
    ]
    scratch_shapes = [
        pltpu.VMEM((B, S), jnp.bfloat16),
        pltpu.VMEM((B, H), jnp.bfloat16),
        pltpu.VMEM((B, H), jnp.bfloat16),
        pltpu.VMEM((B, Ap), jnp.float32),
    ]

    probs, value, act, alp = pl.pallas_call(
        _body,
        grid=(steps,),
        in_specs=in_specs,
        out_specs=out_specs,
        out_shape=out_shape,
        scratch_shapes=scratch_shapes,
    )(state,
      W1, W1, W1, W1, b1.reshape(1, H),
      W2, W2, W2, W2, b2.reshape(1, H),
      W3, W3, W3, W3, b3.reshape(1, A),
      V1, V1, V1, V1, vb1.reshape(1, H),
      V2, V2, V2, V2, vb2.reshape(1, H),
      V3.reshape(1, S), vb3.reshape(1, 1), g)
    return probs, value, act[:, 0], alp[:, 0]
